# baseline (speedup 1.0000x reference)
import jax
import jax.numpy as jnp
from jax import lax
from jax.experimental import pallas as pl
from jax.experimental.pallas import tpu as pltpu
from jax.experimental.pallas import tpu_sc as plsc

_E = 16
_INV_TEMP = 2.0
_HW = 784
_KTC = 608
_KSC = _HW - _KTC
_TC_STEPS = 8
_NC, _NS = 2, 16
_NQ = 4
_QP = _KSC // _NQ
_NR = 11
_NCH = _QP // _NR


def _tc_pool_body(x_ref, o_ref):
    i = pl.program_id(0)
    part = jnp.sum(x_ref[...], axis=0)

    @pl.when(i == 0)
    def _init():
        o_ref[...] = part

    @pl.when(i > 0)
    def _acc():
        o_ref[...] += part


def _sc_pool_body(x_ref, o_ref, buf0, buf1, acc, sem0, sem1):
    w = lax.axis_index("s") * _NC + lax.axis_index("c")
    s = w % 8
    q = w // 8
    r0 = 8 * s
    p0 = _KTC + q * _QP
    bufs = (buf0, buf1)
    sems = (sem0, sem1)

    def _copy(g, buf, sem):
        src = x_ref.at[pl.ds(p0 + g * _NR, _NR), pl.ds(r0, 8), :]
        return pltpu.make_async_copy(src, buf, sem)

    _copy(0, buf0, sem0).start()
    for g in range(_NCH):
        cur = bufs[g % 2]
        if g + 1 < _NCH:
            _copy(g + 1, bufs[(g + 1) % 2], sems[(g + 1) % 2]).start()
        _copy(g, cur, sems[g % 2]).wait()

        for part in range(12):
            base = 16 * part
            cols = [((base + j) // 24, (base + j) % 24) for j in range(16)]
            if g == 0:
                carry0 = tuple(jnp.zeros((16,), jnp.float32) for _ in cols)
            else:
                carry0 = tuple(acc[rr2, pl.ds(16 * k2, 16)]
                               for rr2, k2 in cols)

            def _plane(n, carry):
                return tuple(
                    c + cur[n, rr2, pl.ds(16 * k2, 16)]
                    for c, (rr2, k2) in zip(carry, cols))

            out = lax.fori_loop(0, _NR, _plane, carry0)
            for c, (rr2, k2) in zip(out, cols):
                acc[rr2, pl.ds(16 * k2, 16)] = c
    pltpu.sync_copy(acc, o_ref.at[q, pl.ds(r0, 8), :])


def _combine_body(a_ref, s_ref, w_ref, b_ref, o_ref):
    pooled = (a_ref[...] + jnp.sum(s_ref[...], axis=0)) * (1.0 / _HW)
    logits = lax.dot_general(
        w_ref[...], pooled, (((1,), (1,)), ((), ())),
        preferred_element_type=jnp.float32)
    logits = (logits + b_ref[...]) * _INV_TEMP
    m = jnp.max(logits, axis=0, keepdims=True)
    e = jnp.exp(logits - m)
    o_ref[...] = e / jnp.sum(e, axis=0, keepdims=True)


def kernel(x, W, b):
    B, C = x.shape[0], x.shape[1]
    xt = jnp.transpose(x, (2, 3, 0, 1)).reshape(_HW, B, C)
    b2 = b.reshape(_E, 1)

    part_tc = pl.pallas_call(
        _tc_pool_body,
        grid=(_TC_STEPS,),
        in_specs=[pl.BlockSpec((_KTC // _TC_STEPS, B, C), lambda i: (i, 0, 0))],
        out_specs=pl.BlockSpec((B, C), lambda i: (0, 0)),
        out_shape=jax.ShapeDtypeStruct((B, C), jnp.float32),
    )(xt)

    mesh = plsc.VectorSubcoreMesh(core_axis_name="c", subcore_axis_name="s",
                                  num_cores=_NC, num_subcores=_NS)
    part_sc = pl.kernel(
        _sc_pool_body,
        out_type=jax.ShapeDtypeStruct((_NQ, B, C), jnp.float32),
        mesh=mesh,
        scratch_types=[
            pltpu.VMEM((_NR, 8, 384), jnp.float32),
            pltpu.VMEM((_NR, 8, 384), jnp.float32),
            pltpu.VMEM((8, 384), jnp.float32),
            pltpu.SemaphoreType.DMA,
            pltpu.SemaphoreType.DMA,
        ],
    )(xt)

    out_t = pl.pallas_call(
        _combine_body,
        in_specs=[
            pl.BlockSpec((B, C), lambda: (0, 0)),
            pl.BlockSpec((_NQ, B, C), lambda: (0, 0, 0)),
            pl.BlockSpec((_E, C), lambda: (0, 0)),
            pl.BlockSpec((_E, 1), lambda: (0, 0)),
        ],
        out_specs=pl.BlockSpec((_E, B), lambda: (0, 0)),
        out_shape=jax.ShapeDtypeStruct((_E, B), jnp.float32),
    )(part_tc, part_sc, W, b2)
    return out_t.T


# speedup vs baseline: 1.0722x; 1.0722x over previous
import jax
import jax.numpy as jnp
from jax import lax
from jax.experimental import pallas as pl
from jax.experimental.pallas import tpu as pltpu
from jax.experimental.pallas import tpu_sc as plsc

_E = 16
_INV_TEMP = 2.0
_HW = 784
_KTC = 664
_KSC = _HW - _KTC
_TC_STEPS = 8
_NC, _NS = 2, 16
_NQ = 4
_QP = _KSC // _NQ
_NR = 15
_NCH = _QP // _NR


def _tc_pool_body(x_ref, o_ref):
    i = pl.program_id(0)
    part = jnp.sum(x_ref[...], axis=0)

    @pl.when(i == 0)
    def _init():
        o_ref[...] = part

    @pl.when(i > 0)
    def _acc():
        o_ref[...] += part


def _sc_pool_body(x_ref, o_ref, buf0, buf1, acc, sem0, sem1):
    w = lax.axis_index("s") * _NC + lax.axis_index("c")
    s = w % 8
    q = w // 8
    r0 = 8 * s
    p0 = _KTC + q * _QP
    bufs = (buf0, buf1)
    sems = (sem0, sem1)

    def _copy(g, buf, sem):
        src = x_ref.at[pl.ds(p0 + g * _NR, _NR), pl.ds(r0, 8), :]
        return pltpu.make_async_copy(src, buf, sem)

    _copy(0, buf0, sem0).start()
    for g in range(_NCH):
        cur = bufs[g % 2]
        if g + 1 < _NCH:
            _copy(g + 1, bufs[(g + 1) % 2], sems[(g + 1) % 2]).start()
        _copy(g, cur, sems[g % 2]).wait()

        for part in range(12):
            base = 16 * part
            cols = [((base + j) // 24, (base + j) % 24) for j in range(16)]
            if g == 0:
                carry0 = tuple(jnp.zeros((16,), jnp.float32) for _ in cols)
            else:
                carry0 = tuple(acc[rr2, pl.ds(16 * k2, 16)]
                               for rr2, k2 in cols)

            def _plane(n, carry):
                return tuple(
                    c + cur[n, rr2, pl.ds(16 * k2, 16)]
                    for c, (rr2, k2) in zip(carry, cols))

            out = lax.fori_loop(0, _NR, _plane, carry0)
            for c, (rr2, k2) in zip(out, cols):
                acc[rr2, pl.ds(16 * k2, 16)] = c
    pltpu.sync_copy(acc, o_ref.at[q, pl.ds(r0, 8), :])


def _combine_body(a_ref, s_ref, w_ref, b_ref, o_ref):
    pooled = (a_ref[...] + jnp.sum(s_ref[...], axis=0)) * (1.0 / _HW)
    logits = lax.dot_general(
        w_ref[...], pooled, (((1,), (1,)), ((), ())),
        preferred_element_type=jnp.float32)
    logits = (logits + b_ref[...]) * _INV_TEMP
    m = jnp.max(logits, axis=0, keepdims=True)
    e = jnp.exp(logits - m)
    o_ref[...] = e / jnp.sum(e, axis=0, keepdims=True)


def kernel(x, W, b):
    B, C = x.shape[0], x.shape[1]
    xt = jnp.transpose(x, (2, 3, 0, 1)).reshape(_HW, B, C)
    b2 = b.reshape(_E, 1)

    part_tc = pl.pallas_call(
        _tc_pool_body,
        grid=(_TC_STEPS,),
        in_specs=[pl.BlockSpec((_KTC // _TC_STEPS, B, C), lambda i: (i, 0, 0))],
        out_specs=pl.BlockSpec((B, C), lambda i: (0, 0)),
        out_shape=jax.ShapeDtypeStruct((B, C), jnp.float32),
    )(xt)

    mesh = plsc.VectorSubcoreMesh(core_axis_name="c", subcore_axis_name="s",
                                  num_cores=_NC, num_subcores=_NS)
    part_sc = pl.kernel(
        _sc_pool_body,
        out_type=jax.ShapeDtypeStruct((_NQ, B, C), jnp.float32),
        mesh=mesh,
        scratch_types=[
            pltpu.VMEM((_NR, 8, 384), jnp.float32),
            pltpu.VMEM((_NR, 8, 384), jnp.float32),
            pltpu.VMEM((8, 384), jnp.float32),
            pltpu.SemaphoreType.DMA,
            pltpu.SemaphoreType.DMA,
        ],
    )(xt)

    out_t = pl.pallas_call(
        _combine_body,
        in_specs=[
            pl.BlockSpec((B, C), lambda: (0, 0)),
            pl.BlockSpec((_NQ, B, C), lambda: (0, 0, 0)),
            pl.BlockSpec((_E, C), lambda: (0, 0)),
            pl.BlockSpec((_E, 1), lambda: (0, 0)),
        ],
        out_specs=pl.BlockSpec((_E, B), lambda: (0, 0)),
        out_shape=jax.ShapeDtypeStruct((_E, B), jnp.float32),
    )(part_tc, part_sc, W, b2)
    return out_t.T
